# Initial kernel scaffold; baseline (speedup 1.0000x reference)
import jax
import jax.numpy as jnp
from jax.experimental import pallas as pl


def kernel(x, Wr, br, W1, b1, W2, b2):
    raise NotImplementedError("write your pallas kernel here")


# speedup vs baseline: 1.2677x; 1.2677x over previous
import functools

import jax
import jax.numpy as jnp
from jax import lax
from jax.experimental import pallas as pl
from jax.experimental.pallas import tpu as pltpu

T, D, H, O, E, K = 2048, 1024, 2048, 1024, 8, 2
BLK = 256
NB = (T * K) // BLK + E
P = NB * BLK
EP = 128


def _router_body(x_ref, wr_ref, br_ref, tw_ref, ti_ref):
    logits = jnp.dot(x_ref[...], wr_ref[...],
                     preferred_element_type=jnp.float32) + br_ref[...]
    idx = lax.broadcasted_iota(jnp.int32, logits.shape, 1)
    big = jnp.int32(1 << 30)
    m1 = jnp.max(logits, axis=-1, keepdims=True)
    i1 = jnp.min(jnp.where(logits == m1, idx, big), axis=-1, keepdims=True)
    l2 = jnp.where(idx == i1, -jnp.inf, logits)
    m2 = jnp.max(l2, axis=-1, keepdims=True)
    i2 = jnp.min(jnp.where(l2 == m2, idx, big), axis=-1, keepdims=True)
    den = jnp.sum(jnp.exp(logits - m1), axis=-1, keepdims=True)
    w1 = jnp.exp(m1 - m1) / den
    w2 = jnp.exp(m2 - m1) / den
    tw_ref[...] = jnp.where(idx == 0, w1, jnp.where(idx == 1, w2, 0.0))
    ti_ref[...] = jnp.where(idx == 0, i1, jnp.where(idx == 1, i2, 0))


def _router(x, Wr, br):
    wr_p = jnp.zeros((D, EP), jnp.float32).at[:, :E].set(Wr)
    br_p = jnp.full((1, EP), -1e30, jnp.float32).at[0, :E].set(br)
    tw, ti = pl.pallas_call(
        _router_body,
        out_shape=(jax.ShapeDtypeStruct((T, EP), jnp.float32),
                   jax.ShapeDtypeStruct((T, EP), jnp.int32)),
    )(x, wr_p, br_p)
    return tw[:, :K], ti[:, :K]


def _route(top_w, top_i):
    eid = top_i.reshape(-1)
    onehot = (eid[:, None] == jnp.arange(E, dtype=jnp.int32)[None, :])
    onehot = onehot.astype(jnp.int32)
    counts = jnp.sum(onehot, axis=0)
    rank = jnp.take_along_axis(jnp.cumsum(onehot, axis=0) - onehot,
                               eid[:, None], axis=1)[:, 0]
    nb = (counts + BLK - 1) // BLK
    bstart = jnp.concatenate([jnp.zeros((1,), jnp.int32),
                              jnp.cumsum(nb).astype(jnp.int32)])
    dest = bstart[eid] * BLK + rank
    u = bstart[E]
    j = jnp.arange(NB, dtype=jnp.int32)
    e_map = jnp.minimum(
        jnp.sum((j[:, None] >= bstart[1:][None, :]).astype(jnp.int32), axis=1),
        E - 1)
    valid = (j < u).astype(jnp.int32)
    pairtok = jnp.arange(T * K, dtype=jnp.int32) // K
    tok_sorted = (jnp.arange(P, dtype=jnp.int32) % T).at[dest].set(pairtok)
    w_sorted = jnp.zeros((P,), jnp.float32).at[dest].set(top_w.reshape(-1))
    pos = dest.reshape(T, K)
    return tok_sorted, w_sorted, pos, e_map, valid


def _gemm_body(s_ref, xs_ref, w1_ref, b1_ref, w2_ref, b2_ref, ws_ref,
               out_ref):
    i = pl.program_id(0)

    @pl.when(s_ref[NB + i] == 1)
    def _():
        h = jnp.dot(xs_ref[...], w1_ref[0],
                    preferred_element_type=jnp.float32) + b1_ref[0]
        h = jnp.maximum(h, 0.0)
        y = jnp.dot(h, w2_ref[0],
                    preferred_element_type=jnp.float32) + b2_ref[0]
        out_ref[...] = y * ws_ref[...][:, None]


def _grouped_gemm(s, xs, W1, b1, W2, b2, w_sorted):
    grid_spec = pltpu.PrefetchScalarGridSpec(
        num_scalar_prefetch=1,
        grid=(NB,),
        in_specs=[
            pl.BlockSpec((BLK, D), lambda i, s: (i, 0)),
            pl.BlockSpec((1, D, H), lambda i, s: (s[i], 0, 0)),
            pl.BlockSpec((1, 1, H), lambda i, s: (s[i], 0, 0)),
            pl.BlockSpec((1, H, O), lambda i, s: (s[i], 0, 0)),
            pl.BlockSpec((1, 1, O), lambda i, s: (s[i], 0, 0)),
            pl.BlockSpec((BLK,), lambda i, s: (i,)),
        ],
        out_specs=pl.BlockSpec((BLK, O), lambda i, s: (i, 0)),
    )
    return pl.pallas_call(
        _gemm_body,
        grid_spec=grid_spec,
        out_shape=jax.ShapeDtypeStruct((P, O), jnp.float32),
        compiler_params=pltpu.CompilerParams(
            dimension_semantics=("arbitrary",)),
    )(s, xs, W1, b1[:, None, :], W2, b2[:, None, :], w_sorted)


def kernel(x, Wr, br, W1, b1, W2, b2):
    top_w, top_i = _router(x, Wr, br)
    tok_sorted, w_sorted, pos, e_map, valid = _route(top_w, top_i)
    s = jnp.concatenate([e_map, valid])
    xs = x[tok_sorted]
    ys = _grouped_gemm(s, xs, W1, b1, W2, b2, w_sorted)
    return ys[pos[:, 0]] + ys[pos[:, 1]]


# speedup vs baseline: 1.3770x; 1.0862x over previous
import functools

import jax
import jax.numpy as jnp
from jax import lax
from jax.experimental import pallas as pl
from jax.experimental.pallas import tpu as pltpu
from jax.experimental.pallas import tpu_sc as plsc

T, D, H, O, E, K = 2048, 1024, 2048, 1024, 8, 2
BLK = 256
NB = (T * K) // BLK + E
P = NB * BLK
EP = 128

NW = 32
PAIRS_W = (T * K) // NW
VPW = PAIRS_W // 16
TOK_W = T // NW


def _router_body(x_ref, wr_ref, br_ref, tw_ref, ti_ref, sm_ref):
    logits = jnp.dot(x_ref[...], wr_ref[...],
                     preferred_element_type=jnp.float32) + br_ref[...]
    idx = lax.broadcasted_iota(jnp.int32, logits.shape, 1)
    big = jnp.int32(1 << 30)
    m1 = jnp.max(logits, axis=-1, keepdims=True)
    i1 = jnp.min(jnp.where(logits == m1, idx, big), axis=-1, keepdims=True)
    l2 = jnp.where(idx == i1, -jnp.inf, logits)
    m2 = jnp.max(l2, axis=-1, keepdims=True)
    i2 = jnp.min(jnp.where(l2 == m2, idx, big), axis=-1, keepdims=True)
    den = jnp.sum(jnp.exp(logits - m1), axis=-1, keepdims=True)
    w1 = jnp.exp(m1 - m1) / den
    w2 = jnp.exp(m2 - m1) / den
    tw_ref[...] = jnp.where(idx == 0, w1, jnp.where(idx == 1, w2, 0.0))
    ti_ref[...] = jnp.where(idx == 0, i1, jnp.where(idx == 1, i2, 0))

    oh = (idx == i1).astype(jnp.int32) + (idx == i2).astype(jnp.int32)
    counts = jnp.sum(oh, axis=0, keepdims=True)
    nb = (counts + BLK - 1) // BLK
    jl = idx[0:1, :]
    em = jnp.zeros_like(jl)
    bvec = jnp.zeros_like(jl)
    bend_e = jnp.int32(0)
    for e in range(E):
        bend_e = bend_e + nb[0, e]
        em = em + (jl >= bend_e).astype(jnp.int32)
        bvec = bvec + (jl == 64 + e).astype(jnp.int32) * bend_e
    u = bend_e
    in_em = (jl < 32).astype(jnp.int32)
    in_va = ((jl >= 32) & (jl < 64)).astype(jnp.int32)
    sm_ref[...] = (in_em * jnp.minimum(em, E - 1)
                   + in_va * (jl - 32 < u).astype(jnp.int32)
                   + bvec)


def _router(x, Wr, br):
    wr_p = jnp.zeros((D, EP), jnp.float32).at[:, :E].set(Wr)
    br_p = jnp.full((1, EP), -1e30, jnp.float32).at[0, :E].set(br)
    tw, ti, sm = pl.pallas_call(
        _router_body,
        out_shape=(jax.ShapeDtypeStruct((T, EP), jnp.float32),
                   jax.ShapeDtypeStruct((T, EP), jnp.int32),
                   jax.ShapeDtypeStruct((1, EP), jnp.int32)),
    )(x, wr_p, br_p)
    return tw[:, :K].reshape(-1), ti[:, :K].reshape(-1), sm[0]


def _lanes():
    return lax.broadcasted_iota(jnp.int32, (16,), 0)


_GDN = lax.GatherDimensionNumbers(
    offset_dims=(), collapsed_slice_dims=(0,), start_index_map=(0,))


def _vtake(vec, idx):
    return lax.gather(vec, idx[:, None], _GDN, slice_sizes=(1,),
                      mode=lax.GatherScatterMode.PROMISE_IN_BOUNDS)


def _vtotal(vec):
    ln = _lanes()
    for k in (1, 2, 4, 8):
        vec = vec + _vtake(vec, (ln + k) & 15)
    return vec


def _vprefix(vec):
    ln = _lanes()
    for k in (1, 2, 4, 8):
        shifted = _vtake(vec, jnp.maximum(ln - k, 0))
        vec = vec + shifted * (ln >= k).astype(vec.dtype)
    return vec


_sc_mesh = plsc.VectorSubcoreMesh(core_axis_name="c", subcore_axis_name="s")


@functools.partial(
    pl.kernel,
    out_type=(jax.ShapeDtypeStruct((P, D), jnp.float32),
              jax.ShapeDtypeStruct((T * K,), jnp.int32)),
    mesh=_sc_mesh,
    scratch_types=[
        pltpu.VMEM((T * K,), jnp.int32),
        pltpu.VMEM((16,), jnp.int32),
        pltpu.VMEM((16,), jnp.int32),
        pltpu.VMEM((PAIRS_W,), jnp.int32),
        pltpu.VMEM((32,), jnp.int32),
        pltpu.VMEM((32,), jnp.int32),
        pltpu.VMEM((32, D), jnp.float32),
        pltpu.SemaphoreType.DMA,
        pltpu.SemaphoreType.DMA,
    ],
)
def _dispatch(eid_hbm, x_hbm, widv_hbm, sm_hbm, xs_hbm, pos_hbm,
              eid_v, widv_v, bend_v, pos_v, idx0_v, idx1_v, rows_v,
              sem0, sem1):
    cid = lax.axis_index("c")
    sid = lax.axis_index("s")
    wid = sid * 2 + cid
    lanes = _lanes()
    onehot = [(lanes == e).astype(jnp.int32) for e in range(E)]
    pltpu.sync_copy(eid_hbm, eid_v)
    pltpu.sync_copy(widv_hbm.at[pl.ds(wid * 16, 16)], widv_v)
    pltpu.sync_copy(sm_hbm.at[pl.ds(64, 16)], bend_v)
    wv = widv_v[...]
    bendv = bend_v[...]

    zero = jnp.zeros((16,), jnp.int32)
    msv = wv * VPW

    def pre_body(v, carry):
        pa, pb, vcnt = carry
        vec = eid_v[pl.ds(v * 16, 16)]
        q = vec & 3
        enc = ((q == 0).astype(jnp.int32)
               + (q == 1).astype(jnp.int32) * 256
               + (q == 2).astype(jnp.int32) * 65536
               + (q == 3).astype(jnp.int32) * 16777216)
        m = (vcnt < msv).astype(jnp.int32)
        lo = (vec < 4).astype(jnp.int32) * m
        hi = (vec >= 4).astype(jnp.int32) * m
        return pa + enc * lo, pb + enc * hi, vcnt + 1

    pa, pb, _ = lax.fori_loop(0, (T * K) // 16, pre_body, (zero, zero, zero))
    pre = zero
    for e in range(4):
        pre = pre + onehot[e] * _vtotal(
            lax.shift_right_logical(pa, 8 * e) & 255)
        pre = pre + onehot[e + 4] * _vtotal(
            lax.shift_right_logical(pb, 8 * e) & 255)

    bprev = _vtake(bendv, jnp.maximum(lanes - 1, 0)) * \
        (lanes >= 1).astype(jnp.int32)
    base = bprev * BLK + pre

    my_start = wid * VPW
    full15 = jnp.full((16,), 15, jnp.int32)
    run = [_vtake(base, jnp.full((16,), e, jnp.int32)) for e in range(E)]
    for v in range(VPW):
        vec = eid_v[pl.ds((my_start + v) * 16, 16)]
        dest = zero
        for e in range(E):
            mi = (vec == e).astype(jnp.int32)
            inc = _vprefix(mi)
            dest = dest + mi * (run[e] + inc - 1)
            run[e] = run[e] + _vtake(inc, full15)
        pos_v[pl.ds(v * 16, 16)] = dest
    pltpu.sync_copy(pos_v, pos_hbm.at[pl.ds(wid * PAIRS_W, PAIRS_W)])

    evens = (lanes * 2) & 15
    odds = evens + 1
    below8 = (lanes < 8).astype(jnp.int32)
    for half in range(2):
        pltpu.sync_copy(x_hbm.at[pl.ds(wid * TOK_W + half * 32, 32)], rows_v)
        for g in range(2):
            a = pos_v[pl.ds(half * 64 + g * 32, 16)]
            b = pos_v[pl.ds(half * 64 + g * 32 + 16, 16)]
            idx0_v[pl.ds(g * 16, 16)] = (
                _vtake(a, evens) * below8 + _vtake(b, evens) * (1 - below8))
            idx1_v[pl.ds(g * 16, 16)] = (
                _vtake(a, odds) * below8 + _vtake(b, odds) * (1 - below8))
        cp0 = pltpu.async_copy(rows_v, xs_hbm.at[idx0_v], sem0)
        cp1 = pltpu.async_copy(rows_v, xs_hbm.at[idx1_v], sem1)
        cp0.wait()
        cp1.wait()


def _gemm_body(s_ref, xs_ref, w1_ref, b1_ref, w2_ref, b2_ref, out_ref):
    i = pl.program_id(0)

    @pl.when(s_ref[32 + i] == 1)
    def _():
        h = jnp.dot(xs_ref[...], w1_ref[0],
                    preferred_element_type=jnp.float32) + b1_ref[0]
        h = jnp.maximum(h, 0.0)
        y = jnp.dot(h, w2_ref[0],
                    preferred_element_type=jnp.float32) + b2_ref[0]
        out_ref[...] = y


def _grouped_gemm(smap, xs, W1, b1, W2, b2):
    grid_spec = pltpu.PrefetchScalarGridSpec(
        num_scalar_prefetch=1,
        grid=(NB,),
        in_specs=[
            pl.BlockSpec((BLK, D), lambda i, s: (i, 0)),
            pl.BlockSpec((1, D, H), lambda i, s: (s[i], 0, 0)),
            pl.BlockSpec((1, 1, H), lambda i, s: (s[i], 0, 0)),
            pl.BlockSpec((1, H, O), lambda i, s: (s[i], 0, 0)),
            pl.BlockSpec((1, 1, O), lambda i, s: (s[i], 0, 0)),
        ],
        out_specs=pl.BlockSpec((BLK, O), lambda i, s: (i, 0)),
    )
    return pl.pallas_call(
        _gemm_body,
        grid_spec=grid_spec,
        out_shape=jax.ShapeDtypeStruct((P, O), jnp.float32),
        compiler_params=pltpu.CompilerParams(
            dimension_semantics=("arbitrary",)),
    )(smap, xs, W1, b1[:, None, :], W2, b2[:, None, :])


def kernel(x, Wr, br, W1, b1, W2, b2):
    w_flat, eid_flat, sm = _router(x, Wr, br)
    onehot = (eid_flat[:, None] ==
              jnp.arange(E, dtype=jnp.int32)[None, :]).astype(jnp.int32)
    rank = jnp.take_along_axis(jnp.cumsum(onehot, axis=0) - onehot,
                               eid_flat[:, None], axis=1)[:, 0]
    bstart = jnp.concatenate([jnp.zeros((1,), jnp.int32), sm[64:64 + E]])
    dest = bstart[eid_flat] * BLK + rank
    pairtok = jnp.arange(T * K, dtype=jnp.int32) // K
    tok_sorted = (jnp.arange(P, dtype=jnp.int32) % T).at[dest].set(pairtok)
    xs = x[tok_sorted]
    ys = _grouped_gemm(sm[:64], xs, W1, b1, W2, b2)
    pp = dest.reshape(T, K)
    ww = w_flat.reshape(T, K)
    return ys[pp[:, 0]] * ww[:, :1] + ys[pp[:, 1]] * ww[:, 1:]
